# baseline (speedup 1.0000x reference)
import functools

import jax
import jax.numpy as jnp
from jax import lax
from jax.experimental import pallas as pl
from jax.experimental.pallas import tpu as pltpu
from jax.experimental.pallas import tpu_sc as plsc

L = 16
NC = 2
NS = 16
NW = NC * NS
B = 16384
D = 64
BPW = B // NW
CHUNK = 128
NCH = BPW // CHUNK


def _sc_body(feat_hbm, lab_hbm, centers_hbm, out_hbm,
             idx_v, rows_v, feat_v, acc_v, gsem, fsem):
    wid = lax.axis_index("s") * NC + lax.axis_index("c")
    base = wid * BPW

    pltpu.sync_copy(lab_hbm.at[pl.ds(base, BPW)], idx_v)

    fcopy = pltpu.async_copy(feat_hbm.at[pl.ds(base, BPW)], feat_v, fsem)
    gcopies = []
    for j in range(NCH):
        gcopies.append(pltpu.async_copy(
            centers_hbm.at[idx_v.at[pl.ds(j * CHUNK, CHUNK)]],
            rows_v.at[pl.ds(j * CHUNK, CHUNK)], gsem))
    fcopy.wait()
    for c in gcopies:
        c.wait()

    def row_body(i, accs):
        a0, a1, a2, a3 = accs
        f0 = feat_v[i, pl.ds(0 * L, L)]
        f1 = feat_v[i, pl.ds(1 * L, L)]
        f2 = feat_v[i, pl.ds(2 * L, L)]
        f3 = feat_v[i, pl.ds(3 * L, L)]
        c0 = rows_v[i, pl.ds(0 * L, L)]
        c1 = rows_v[i, pl.ds(1 * L, L)]
        c2 = rows_v[i, pl.ds(2 * L, L)]
        c3 = rows_v[i, pl.ds(3 * L, L)]
        d0 = f0 - c0
        d1 = f1 - c1
        d2 = f2 - c2
        d3 = f3 - c3
        return (a0 + d0 * d0, a1 + d1 * d1, a2 + d2 * d2, a3 + d3 * d3)

    zero = jnp.zeros((L,), jnp.float32)
    a0, a1, a2, a3 = lax.fori_loop(0, BPW, row_body, (zero, zero, zero, zero))
    acc_v[...] = (a0 + a1) + (a2 + a3)
    pltpu.sync_copy(acc_v, out_hbm.at[wid])


@functools.partial(
    pl.kernel,
    out_type=jax.ShapeDtypeStruct((NW, L), jnp.float32),
    mesh=plsc.VectorSubcoreMesh(core_axis_name="c", subcore_axis_name="s"),
    compiler_params=pltpu.CompilerParams(use_tc_tiling_on_sc=False),
    scratch_types=[
        pltpu.VMEM((BPW,), jnp.int32),
        pltpu.VMEM((BPW, D), jnp.float32),
        pltpu.VMEM((BPW, D), jnp.float32),
        pltpu.VMEM((L,), jnp.float32),
        pltpu.SemaphoreType.DMA,
        pltpu.SemaphoreType.DMA,
    ],
)
def _centerloss_partials(feat_hbm, lab_hbm, centers_hbm, out_hbm,
                         idx_v, rows_v, feat_v, acc_v, gsem, fsem):
    _sc_body(feat_hbm, lab_hbm, centers_hbm, out_hbm,
             idx_v, rows_v, feat_v, acc_v, gsem, fsem)


def kernel(feature, label, centers, batch_size):
    partials = _centerloss_partials(feature, label.astype(jnp.int32), centers)
    return jnp.sum(partials) / 2.0 / batch_size


# speedup vs baseline: 2.1138x; 2.1138x over previous
import functools

import jax
import jax.numpy as jnp
from jax import lax
from jax.experimental import pallas as pl
from jax.experimental.pallas import tpu as pltpu
from jax.experimental.pallas import tpu_sc as plsc

L = 16
NC = 2
NS = 16
NW = NC * NS
B = 16384
D = 64
V = 100000
HALF = B // 2


def _sc_body(feat_hbm, lab_hbm, centers_hbm, out_hbm,
             row_v, feat_v, lab_v, acc_v, rsem, fsem, lsem):
    wid = lax.axis_index("s") * NC + lax.axis_index("c")

    centers3 = centers_hbm.reshape(D // 8, 8, V)
    feat3 = feat_hbm.reshape(D // 8, 8, B)

    def sweep(d, acc):
        rcopy = pltpu.async_copy(centers3.at[d // 8, d % 8], row_v, rsem)
        for h in range(2):
            lcopy = pltpu.async_copy(lab_hbm.at[pl.ds(h * HALF, HALF)],
                                     lab_v, lsem)
            fcopy = pltpu.async_copy(feat3.at[d // 8, d % 8,
                                              pl.ds(h * HALF, HALF)],
                                     feat_v, fsem)
            if h == 0:
                rcopy.wait()
            lcopy.wait()
            fcopy.wait()

            def step(i, accs):
                a0, a1 = accs
                i0 = lab_v[pl.ds(2 * i * L, L)]
                i1 = lab_v[pl.ds((2 * i + 1) * L, L)]
                f0 = feat_v[pl.ds(2 * i * L, L)]
                f1 = feat_v[pl.ds((2 * i + 1) * L, L)]
                c0 = plsc.load_gather(row_v, [i0])
                c1 = plsc.load_gather(row_v, [i1])
                d0 = f0 - c0
                d1 = f1 - c1
                return (a0 + d0 * d0, a1 + d1 * d1)

            acc = lax.fori_loop(0, HALF // (2 * L), step, acc)
        return acc

    zero = jnp.zeros((L,), jnp.float32)
    a0, a1 = sweep(wid, (zero, zero))
    a0, a1 = sweep(wid + NW, (a0, a1))
    acc_v[...] = a0 + a1
    pltpu.sync_copy(acc_v, out_hbm.at[pl.ds(wid * L, L)])


@functools.partial(
    pl.kernel,
    out_type=jax.ShapeDtypeStruct((NW * L,), jnp.float32),
    mesh=plsc.VectorSubcoreMesh(core_axis_name="c", subcore_axis_name="s"),
    compiler_params=pltpu.CompilerParams(use_tc_tiling_on_sc=True,
                                        needs_layout_passes=False),
    scratch_types=[
        pltpu.VMEM((V,), jnp.float32),
        pltpu.VMEM((HALF,), jnp.float32),
        pltpu.VMEM((HALF,), jnp.int32),
        pltpu.VMEM((L,), jnp.float32),
        pltpu.SemaphoreType.DMA,
        pltpu.SemaphoreType.DMA,
        pltpu.SemaphoreType.DMA,
    ],
)
def _centerloss_partials(feat_hbm, lab_hbm, centers_hbm, out_hbm,
                         row_v, feat_v, lab_v, acc_v, rsem, fsem, lsem):
    _sc_body(feat_hbm, lab_hbm, centers_hbm, out_hbm,
             row_v, feat_v, lab_v, acc_v, rsem, fsem, lsem)


def kernel(feature, label, centers, batch_size):
    partials = _centerloss_partials(feature.T, label.astype(jnp.int32),
                                    centers.T)
    return jnp.sum(partials) / 2.0 / batch_size


# speedup vs baseline: 2.3631x; 1.1179x over previous
import functools

import jax
import jax.numpy as jnp
from jax import lax
from jax.experimental import pallas as pl
from jax.experimental.pallas import tpu as pltpu
from jax.experimental.pallas import tpu_sc as plsc

L = 16
NC = 2
NS = 16
NW = NC * NS
B = 16384
D = 64
V = 100000
CH = 4096
NCHK = B // CH
UN = 4


def _sc_body(feat_hbm, lab_hbm, centers_hbm, out_hbm,
             row_v, f0_v, f1_v, lab_v, acc_v, rsem, fsem, lsem):
    wid = lax.axis_index("s") * NC + lax.axis_index("c")

    centers3 = centers_hbm.reshape(D // 8, 8, V)
    feat3 = feat_hbm.reshape(D // 8, 8, B)
    fbufs = (f0_v, f1_v)

    def row_copy(d):
        return pltpu.async_copy(centers3.at[d // 8, d % 8], row_v, rsem)

    def feat_copy(d, c, buf):
        return pltpu.async_copy(
            feat3.at[d // 8, d % 8, pl.ds(c * CH, CH)], fbufs[buf], fsem)

    def chunk_compute(cbase, buf, acc):
        fb = fbufs[buf]

        def step(i, accs):
            acc = list(accs)
            for u in range(UN):
                idx = lab_v[pl.ds(cbase + (UN * i + u) * L, L)]
                f = fb[pl.ds((UN * i + u) * L, L)]
                c = plsc.load_gather(row_v, [idx])
                dd = f - c
                acc[u] = acc[u] + dd * dd
            return tuple(acc)

        return lax.fori_loop(0, CH // (UN * L), step, acc)

    d1, d2 = wid, wid + NW
    rcopy = row_copy(d1)
    lcopy = pltpu.async_copy(lab_hbm, lab_v, lsem)
    fcopy = feat_copy(d1, 0, 0)
    lcopy.wait()

    zero = jnp.zeros((L,), jnp.float32)
    acc = (zero,) * UN
    for k, d in enumerate((d1, d2)):
        rcopy.wait()
        for c in range(NCHK):
            nxt_fcopy = None
            if c + 1 < NCHK:
                nxt_fcopy = feat_copy(d, c + 1, (c + 1) % 2)
            elif k == 0:
                nxt_fcopy = feat_copy(d2, 0, (c + 1) % 2)
            fcopy.wait()
            acc = chunk_compute(c * CH, c % 2, acc)
            if c == NCHK - 1 and k == 0:
                rcopy = row_copy(d2)
            fcopy = nxt_fcopy

    acc_v[...] = (acc[0] + acc[1]) + (acc[2] + acc[3])
    pltpu.sync_copy(acc_v, out_hbm.at[pl.ds(wid * L, L)])


@functools.partial(
    pl.kernel,
    out_type=jax.ShapeDtypeStruct((NW * L,), jnp.float32),
    mesh=plsc.VectorSubcoreMesh(core_axis_name="c", subcore_axis_name="s"),
    compiler_params=pltpu.CompilerParams(use_tc_tiling_on_sc=True,
                                        needs_layout_passes=False),
    scratch_types=[
        pltpu.VMEM((V,), jnp.float32),
        pltpu.VMEM((CH,), jnp.float32),
        pltpu.VMEM((CH,), jnp.float32),
        pltpu.VMEM((B,), jnp.int32),
        pltpu.VMEM((L,), jnp.float32),
        pltpu.SemaphoreType.DMA,
        pltpu.SemaphoreType.DMA,
        pltpu.SemaphoreType.DMA,
    ],
)
def _centerloss_partials(feat_hbm, lab_hbm, centers_hbm, out_hbm,
                         row_v, f0_v, f1_v, lab_v, acc_v, rsem, fsem, lsem):
    _sc_body(feat_hbm, lab_hbm, centers_hbm, out_hbm,
             row_v, f0_v, f1_v, lab_v, acc_v, rsem, fsem, lsem)


def kernel(feature, label, centers, batch_size):
    partials = _centerloss_partials(feature.T, label.astype(jnp.int32),
                                    centers.T)
    return jnp.sum(partials) / 2.0 / batch_size
